# baseline (speedup 1.0000x reference)
import functools

import jax
import jax.numpy as jnp
from jax.experimental import pallas as pl
from jax.experimental.pallas import tpu as pltpu
from jax.experimental.pallas import tpu_sc as plsc

_TEMP = 0.1
_TOPK = 32
_N = 4096
_D = 256
_BLK = 256
_NW = 32
_RPW = _N // _NW
_LANES = 16
_GROW = 8
_NGRP = _RPW // _GROW
_HALF = _N // 2
_SLOTS = 192
_NEG = float("-inf")


def _sim_kernel(xb_ref, xf_ref, sim_ref, lb_ref):
    raw = jax.lax.dot_general(
        xb_ref[...], xf_ref[...],
        dimension_numbers=(((1,), (1,)), ((), ())),
        preferred_element_type=jnp.float32,
    )
    sim_ref[...] = raw

    m = raw[:, 0:128]
    for c in range(1, 32):
        m = jnp.maximum(m, raw[:, c * 128:(c + 1) * 128])

    def knock_out(_, s):
        mx = jnp.max(s, axis=1, keepdims=True)
        return jnp.where(s >= mx, -jnp.inf, s)

    m = jax.lax.fori_loop(0, _TOPK - 1, knock_out, m)
    lb_ref[...] = jnp.max(m, axis=1, keepdims=True)


def _adj_kernel(sim_ref, tcol_ref, trow_ref, out_ref):
    raw = sim_ref[...]
    in_row = (raw >= tcol_ref[...]).astype(jnp.float32)
    in_col = (raw >= trow_ref[...]).astype(jnp.float32)
    out_ref[...] = (raw / jnp.float32(_TEMP)) * ((in_row + in_col) * 0.5)


def _u32_key(v):
    bu = plsc.bitcast(v, jnp.uint32)
    flip = jnp.where(bu >= jnp.uint32(0x80000000),
                     jnp.uint32(0xFFFFFFFF), jnp.uint32(0x80000000))
    return bu ^ flip


def _inv_key(k):
    flip = jnp.where(k >= jnp.uint32(0x80000000),
                     jnp.uint32(0x80000000), jnp.uint32(0xFFFFFFFF))
    return plsc.bitcast(k ^ flip, jnp.float32)


def _thr_sc_body(sim_hbm, lb_hbm, thr_hbm, buf0, buf1, cand_v, key_v,
                 lb_v, thr_v, sem0, sem1):
    wid = jax.lax.axis_index("s") * 2 + jax.lax.axis_index("c")
    base = wid * _RPW

    iota = jax.lax.iota(jnp.int32, _LANES)
    rowsel = jnp.where(iota < _GROW, iota, 15 - iota)
    colaux = (iota >> 3) * _HALF
    lanebase = iota * _SLOTS
    neg16 = jnp.full((_LANES,), _NEG, jnp.float32)
    ones = jnp.full((_LANES,), 1, jnp.int32)
    zero16 = jnp.full((_LANES,), 0, jnp.int32)

    pltpu.async_copy(lb_hbm.at[pl.ds(base, _RPW)], lb_v, sem0).wait()
    pltpu.async_copy(sim_hbm.at[pl.ds(base, _GROW)], buf0, sem0)

    def do_group(g, _):
        even = jax.lax.rem(g, 2) == 0

        @pl.when(g + 1 < _NGRP)
        def _():
            nxt = pl.ds(base + (g + 1) * _GROW, _GROW)

            @pl.when(even)
            def _():
                pltpu.async_copy(sim_hbm.at[nxt], buf1, sem1)

            @pl.when(jnp.logical_not(even))
            def _():
                pltpu.async_copy(sim_hbm.at[nxt], buf0, sem0)

        def run(buf, sem):
            pltpu.make_async_copy(
                sim_hbm.at[pl.ds(base + g * _GROW, _GROW)], buf, sem).wait()
            L16 = plsc.load_gather(lb_v, [g * _GROW + rowsel])

            def prefill(p, _):
                for u in range(4):
                    cand_v[pl.ds((4 * p + u) * _LANES, _LANES)] = neg16
                return 0

            jax.lax.fori_loop(0, _LANES * _SLOTS // (4 * _LANES), prefill, 0)

            def scan(j, cidx):
                for u in range(4):
                    col = colaux + (4 * j + u)
                    v = plsc.load_gather(buf, [rowsel, col])
                    m = v >= L16
                    addr = lanebase + jnp.minimum(
                        cidx, jnp.int32(_SLOTS - 1))
                    plsc.store_scatter(cand_v, [addr], v, mask=m)
                    cidx = cidx + jnp.where(m, ones, zero16)
                return cidx

            cidx = jax.lax.fori_loop(0, _HALF // 4, scan, zero16)

            def to_key(p, _):
                for u in range(4):
                    s = pl.ds((4 * p + u) * _LANES, _LANES)
                    key_v[s] = plsc.bitcast(_u32_key(cand_v[s]), jnp.int32)
                return 0

            jax.lax.fori_loop(0, _LANES * _SLOTS // (4 * _LANES), to_key, 0)

            maxc = jnp.max(cidx)
            lo0 = _u32_key(L16)
            hi0 = jnp.full((_LANES,), jnp.uint32(0xFF7FFFFF))

            def bisect_cand(_b, lohi):
                lo, hi = lohi
                mid = lo + ((hi - lo + jnp.uint32(1)) >> 1)

                def count(k, c):
                    kv = plsc.bitcast(
                        plsc.load_gather(key_v, [lanebase + k]), jnp.uint32)
                    return c + jnp.where(kv >= mid, ones, zero16)

                c = jax.lax.fori_loop(0, maxc, count, zero16)
                c = c + jax.lax.rev(c, dimensions=(0,))
                good = c >= jnp.int32(_TOPK)
                return (jnp.where(good, mid, lo),
                        jnp.where(good, hi, mid - jnp.uint32(1)))

            def bisect_full(_b, lohi):
                lo, hi = lohi
                mid = lo + ((hi - lo + jnp.uint32(1)) >> 1)

                def count(j, c):
                    kv = _u32_key(plsc.load_gather(buf, [rowsel, colaux + j]))
                    return c + jnp.where(kv >= mid, ones, zero16)

                c = jax.lax.fori_loop(0, _HALF, count, zero16)
                c = c + jax.lax.rev(c, dimensions=(0,))
                good = c >= jnp.int32(_TOPK)
                return (jnp.where(good, mid, lo),
                        jnp.where(good, hi, mid - jnp.uint32(1)))

            def from_cand():
                lo, _hi = jax.lax.fori_loop(0, 32, bisect_cand, (lo0, hi0))
                return _inv_key(lo)

            def from_full():
                lo, _hi = jax.lax.fori_loop(0, 32, bisect_full, (lo0, hi0))
                return _inv_key(lo)

            return jax.lax.cond(maxc <= jnp.int32(_SLOTS),
                                from_cand, from_full)

        t16 = jax.lax.cond(even, lambda: run(buf0, sem0),
                           lambda: run(buf1, sem1))
        plsc.store_compressed(thr_v.at[pl.ds(g * _GROW, _LANES)], t16,
                              mask=iota < _GROW)
        return 0

    jax.lax.fori_loop(0, _NGRP, do_group, 0)
    pltpu.sync_copy(thr_v.at[pl.ds(0, _RPW)], thr_hbm.at[pl.ds(base, _RPW)])


_thr_sc = functools.partial(
    pl.kernel,
    out_type=jax.ShapeDtypeStruct((_N,), jnp.float32),
    mesh=plsc.VectorSubcoreMesh(core_axis_name="c", subcore_axis_name="s"),
    compiler_params=pltpu.CompilerParams(needs_layout_passes=False),
    scratch_types=[
        pltpu.VMEM((_GROW, _N), jnp.float32),
        pltpu.VMEM((_GROW, _N), jnp.float32),
        pltpu.VMEM((_LANES * _SLOTS,), jnp.float32),
        pltpu.VMEM((_LANES * _SLOTS,), jnp.int32),
        pltpu.VMEM((_RPW,), jnp.float32),
        pltpu.VMEM((_RPW + _LANES,), jnp.float32),
        pltpu.SemaphoreType.DMA,
        pltpu.SemaphoreType.DMA,
    ],
)(_thr_sc_body)


def kernel(x):
    nblk = _N // _BLK
    sim, lb = pl.pallas_call(
        _sim_kernel,
        grid=(nblk,),
        in_specs=[
            pl.BlockSpec((_BLK, _D), lambda i: (i, 0)),
            pl.BlockSpec((_N, _D), lambda i: (0, 0)),
        ],
        out_specs=[
            pl.BlockSpec((_BLK, _N), lambda i: (i, 0)),
            pl.BlockSpec((_BLK, 1), lambda i: (i, 0)),
        ],
        out_shape=[
            jax.ShapeDtypeStruct((_N, _N), jnp.float32),
            jax.ShapeDtypeStruct((_N, 1), jnp.float32),
        ],
    )(x, x)

    thr = _thr_sc(sim, lb.reshape(_N))
    tcol = thr.reshape(_N, 1)
    trow = thr.reshape(1, _N)

    adj = pl.pallas_call(
        _adj_kernel,
        grid=(nblk,),
        in_specs=[
            pl.BlockSpec((_BLK, _N), lambda i: (i, 0)),
            pl.BlockSpec((_BLK, 1), lambda i: (i, 0)),
            pl.BlockSpec((1, _N), lambda i: (0, 0)),
        ],
        out_specs=pl.BlockSpec((_BLK, _N), lambda i: (i, 0)),
        out_shape=jax.ShapeDtypeStruct((_N, _N), jnp.float32),
    )(sim, tcol, trow)
    return adj
